# baseline (speedup 1.0000x reference)
import jax
import jax.numpy as jnp
from jax import lax
from jax.experimental import pallas as pl
from jax.experimental.pallas import tpu as pltpu
from jax.experimental.pallas import tpu_sc as plsc

_L = 16385
_D = 64
_NREL = 1000
_CHUNK = 128
_NW = 32
_CPW = 4
_ROWS_PW = _CHUNK * _CPW
_PAD = _NW * _ROWS_PW + _CHUNK


def _sc_body(path_hbm, table_hbm, out_hbm, pbuf, cidx, rows, sem_g, sem_w):
    nc = 2
    wid = lax.axis_index("s") * nc + lax.axis_index("c")
    off = (lax.iota(jnp.int32, 16) & 1) * _NREL

    base = wid * _ROWS_PW
    pltpu.sync_copy(path_hbm.at[pl.ds(base, _ROWS_PW)], pbuf)
    for j in range(_CPW):
        cj = cidx.at[j]
        for k in range(_CHUNK // 16):
            cj[pl.ds(k * 16, 16)] = pbuf[pl.ds(j * _CHUNK + k * 16, 16)] + off
    gathers = [
        pltpu.async_copy(
            table_hbm.at[cidx.at[j]],
            rows.at[pl.ds(j * _CHUNK, _CHUNK)],
            sem_g,
        )
        for j in range(_CPW)
    ]
    writes = []
    for j in range(_CPW):
        gathers[j].wait()
        writes.append(
            pltpu.async_copy(
                rows.at[pl.ds(j * _CHUNK, _CHUNK)],
                out_hbm.at[pl.ds(base + j * _CHUNK, _CHUNK)],
                sem_w,
            )
        )

    @pl.when(wid == 0)
    def _():
        tbase = _NW * _ROWS_PW
        pltpu.sync_copy(path_hbm.at[pl.ds(tbase, _CHUNK)],
                        pbuf.at[pl.ds(0, _CHUNK)])
        cj = cidx.at[0]
        for k in range(_CHUNK // 16):
            s = pl.ds(k * 16, 16)
            cj[s] = pbuf[s] + off
        pltpu.async_copy(
            table_hbm.at[cidx.at[0]],
            rows.at[pl.ds(0, _CHUNK)],
            sem_g,
        ).wait()
        pltpu.async_copy(
            rows.at[pl.ds(0, _CHUNK)],
            out_hbm.at[pl.ds(tbase, _CHUNK)],
            sem_w,
        ).wait()

    for w in writes:
        w.wait()


def _tc_relayout(x_ref, o_ref):
    x = x_ref[...]
    z = jnp.stack([x[:, :_D], x[:, _D:]], axis=1).reshape(512, _D)
    o_ref[...] = z.T


def kernel(path, W_ent, W_rel):
    table = jnp.concatenate([W_ent[:_NREL], W_rel[:_NREL]], axis=0)
    p = jnp.zeros((_PAD,), jnp.int32).at[:_L].set(path.astype(jnp.int32))
    mesh = plsc.VectorSubcoreMesh(core_axis_name="c", subcore_axis_name="s")
    sc_out = pl.kernel(
        _sc_body,
        mesh=mesh,
        compiler_params=pltpu.CompilerParams(use_tc_tiling_on_sc=False),
        out_type=jax.ShapeDtypeStruct((_PAD, _D), jnp.float32),
        scratch_types=[
            pltpu.VMEM((_ROWS_PW,), jnp.int32),
            pltpu.VMEM((_CPW, _CHUNK), jnp.int32),
            pltpu.VMEM((_ROWS_PW, _D), jnp.float32),
            pltpu.SemaphoreType.DMA,
            pltpu.SemaphoreType.DMA,
        ],
    )(p, table)

    x = sc_out.reshape(_PAD // 2, 2 * _D)
    yt = pl.pallas_call(
        _tc_relayout,
        grid=(33,),
        in_specs=[pl.BlockSpec((256, 2 * _D), lambda b: (b, 0))],
        out_specs=pl.BlockSpec((_D, 512), lambda b: (0, b)),
        out_shape=jax.ShapeDtypeStruct((_D, _L), jnp.float32),
    )(x)
    return yt.T
